# baseline (speedup 1.0000x reference)
import functools

import jax
import jax.numpy as jnp
from jax import lax
from jax.experimental import pallas as pl
from jax.experimental.pallas import tpu as pltpu
from jax.experimental.pallas import tpu_sc as plsc

B, T = 4096, 50
NUM_WORDS = T // 2
CB_SIZE, CB_DIM, LLM_DIM = 256, 512, 768
L = 16
T_PAD = 64
NPC = CB_DIM // (2 * L)
STAGE_ROWS = 16

_MASK_HI = -65536
_ROUND = 0x8000


def _sc_words_body(nrows, idx_hbm, cb_hbm, words_hbm, pooled_hbm,
                   cb_p, temp_v, idx_v, words_v0, words_v1,
                   pooled_v0, pooled_v1, sem0, sem1):
    wid = lax.axis_index("s") * 2 + lax.axis_index("c")
    rows_per_w = nrows // 32
    base = wid * rows_per_w

    def stage_piece(p, carry):
        pltpu.sync_copy(cb_hbm.at[pl.ds(p * STAGE_ROWS, STAGE_ROWS)], temp_v)

        def pack_row(r, c2):
            for j in range(NPC):
                lo = temp_v[r, pl.ds(j * 2 * L, L)] * 0.5
                hi = temp_v[r, pl.ds(j * 2 * L + L, L)] * 0.5
                lo_b = lax.shift_right_logical(
                    plsc.bitcast(lo, jnp.int32) + _ROUND, 16)
                hi_b = (plsc.bitcast(hi, jnp.int32) + _ROUND) & _MASK_HI
                cb_p[p * STAGE_ROWS + r, pl.ds(j * L, L)] = hi_b | lo_b
            return c2

        lax.fori_loop(0, STAGE_ROWS, pack_row, 0)
        return carry

    lax.fori_loop(0, CB_SIZE // STAGE_ROWS, stage_piece, 0)
    pltpu.sync_copy(idx_hbm.at[pl.ds(base, rows_per_w)], idx_v)

    def one_row(r, words_b, pooled_b, sem, first):
        row = base + r
        @pl.when(jnp.logical_not(first))
        def _():
            pltpu.make_async_copy(words_b, words_hbm.at[row], sem).wait()
            pltpu.make_async_copy(pooled_b, pooled_hbm.at[row], sem).wait()

        ivs = [idx_v[r, pl.ds(k * L, L)] for k in range(T_PAD // L)]
        ids = [ivs[t // L][t % L] for t in range(T)]

        @plsc.parallel_loop(0, NPC)
        def pc_body(pc):
            slp = pl.ds(pc * L, L)
            sums = []
            for w in range(NUM_WORDS):
                g0 = cb_p[ids[2 * w], slp]
                g1 = cb_p[ids[2 * w + 1], slp]
                a = plsc.bitcast(g0, jnp.bfloat16)
                b = plsc.bitcast(g1, jnp.bfloat16)
                sums.append(a + b)
            acc_lo = [None] * 4
            acc_hi = [None] * 4
            for w in range(NUM_WORDS):
                sb = plsc.bitcast(sums[w], jnp.int32)
                lo = plsc.bitcast(lax.shift_left(sb, 16), jnp.float32)
                hi = plsc.bitcast(sb & _MASK_HI, jnp.float32)
                words_b[w, pl.ds(pc * 2 * L, L)] = lo
                words_b[w, pl.ds(pc * 2 * L + L, L)] = hi
                k = w % 4
                acc_lo[k] = lo if acc_lo[k] is None else acc_lo[k] + lo
                acc_hi[k] = hi if acc_hi[k] is None else acc_hi[k] + hi
            pl_lo = ((acc_lo[0] + acc_lo[1]) + (acc_lo[2] + acc_lo[3]))
            pl_hi = ((acc_hi[0] + acc_hi[1]) + (acc_hi[2] + acc_hi[3]))
            pooled_b[pl.ds(pc * 2 * L, L)] = pl_lo * (1.0 / NUM_WORDS)
            pooled_b[pl.ds(pc * 2 * L + L, L)] = pl_hi * (1.0 / NUM_WORDS)

        pltpu.async_copy(words_b, words_hbm.at[row], sem)
        pltpu.async_copy(pooled_b, pooled_hbm.at[row], sem)

    def pair_body(i, carry):
        one_row(2 * i, words_v0, pooled_v0, sem0, i == 0)
        one_row(2 * i + 1, words_v1, pooled_v1, sem1, i == 0)
        return carry

    lax.fori_loop(0, rows_per_w // 2, pair_body, 0)
    last = base + rows_per_w - 2
    pltpu.make_async_copy(words_v0, words_hbm.at[last], sem0).wait()
    pltpu.make_async_copy(pooled_v0, pooled_hbm.at[last], sem0).wait()
    pltpu.make_async_copy(words_v1, words_hbm.at[last + 1], sem1).wait()
    pltpu.make_async_copy(pooled_v1, pooled_hbm.at[last + 1], sem1).wait()


def _sc_words(char_indices, char_codebook):
    nrows = char_indices.shape[0]
    mesh = plsc.VectorSubcoreMesh(core_axis_name="c", subcore_axis_name="s")
    f = pl.kernel(
        functools.partial(_sc_words_body, nrows),
        out_type=[
            jax.ShapeDtypeStruct((nrows, NUM_WORDS, CB_DIM), jnp.float32),
            jax.ShapeDtypeStruct((nrows, CB_DIM), jnp.float32),
        ],
        mesh=mesh,
        scratch_types=[
            pltpu.VMEM((CB_SIZE, CB_DIM // 2), jnp.int32),
            pltpu.VMEM((STAGE_ROWS, CB_DIM), jnp.float32),
            pltpu.VMEM((nrows // 32, T_PAD), jnp.int32),
            pltpu.VMEM((NUM_WORDS, CB_DIM), jnp.float32),
            pltpu.VMEM((NUM_WORDS, CB_DIM), jnp.float32),
            pltpu.VMEM((CB_DIM,), jnp.float32),
            pltpu.VMEM((CB_DIM,), jnp.float32),
            pltpu.SemaphoreType.DMA,
            pltpu.SemaphoreType.DMA,
        ],
        compiler_params=pltpu.CompilerParams(needs_layout_passes=False),
    )
    return f(char_indices, char_codebook)


def _tc_loss_body(pooled_ref, text_ref, w1_ref, b1_ref, w2_ref, b2_ref,
                  out_ref):
    pooled = pooled_ref[...]
    h = jnp.dot(pooled, w1_ref[...], preferred_element_type=jnp.float32)
    h = jnp.maximum(h + b1_ref[...][None, :], 0.0)
    hbar = jnp.mean(h, axis=0, keepdims=True)
    proj = jnp.dot(hbar, w2_ref[...], preferred_element_type=jnp.float32)
    proj = proj + b2_ref[...][None, :]
    tbar = jnp.mean(text_ref[...], axis=0, keepdims=True)
    d = proj - tbar
    out_ref[...] = jnp.reshape(jnp.mean(d * d), (1, 1))


def _tc_loss(pooled, text_embeddings, W1, b1, W2, b2):
    return pl.pallas_call(
        _tc_loss_body,
        out_shape=jax.ShapeDtypeStruct((1, 1), jnp.float32),
    )(pooled, text_embeddings, W1, b1, W2, b2)


N_CHUNKS = 4


def kernel(char_indices, text_embeddings, char_codebook, W1, b1, W2, b2):
    idx = jnp.pad(char_indices.astype(jnp.int32), ((0, 0), (0, T_PAD - T)))
    bc = B // N_CHUNKS
    parts = [_sc_words(lax.slice_in_dim(idx, k * bc, (k + 1) * bc, axis=0),
                       char_codebook)
             for k in range(N_CHUNKS)]
    words = jnp.concatenate([p[0] for p in parts], axis=0)
    pooled = jnp.concatenate([p[1] for p in parts], axis=0)
    loss = _tc_loss(pooled, text_embeddings, W1, b1, W2, b2)
    return words, loss[0, 0]


# speedup vs baseline: 1.9590x; 1.9590x over previous
import functools

import jax
import jax.numpy as jnp
from jax import lax
from jax.experimental import pallas as pl
from jax.experimental.pallas import tpu as pltpu
from jax.experimental.pallas import tpu_sc as plsc

B, T = 4096, 50
NUM_WORDS = T // 2
CB_SIZE, CB_DIM, LLM_DIM = 256, 512, 768
L = 16
T_PAD = 64
NPC = CB_DIM // (2 * L)
STAGE_ROWS = 16

_MASK_HI = -65536
_ROUND = 0x8000


def _sc_words_body(nrows, idx_hbm, cb_hbm, words_hbm, pooled_hbm,
                   cb_p, temp_v, idx_v, words_v0, words_v1,
                   pooled_v0, pooled_v1, sem0, sem1):
    wid = lax.axis_index("s") * 2 + lax.axis_index("c")
    rows_per_w = nrows // 32
    base = wid * rows_per_w

    def stage_piece(p, carry):
        pltpu.sync_copy(cb_hbm.at[pl.ds(p * STAGE_ROWS, STAGE_ROWS)], temp_v)

        def pack_row(r, c2):
            for j in range(NPC):
                lo = temp_v[r, pl.ds(j * 2 * L, L)] * 0.5
                hi = temp_v[r, pl.ds(j * 2 * L + L, L)] * 0.5
                lo_b = lax.shift_right_logical(
                    plsc.bitcast(lo, jnp.int32) + _ROUND, 16)
                hi_b = (plsc.bitcast(hi, jnp.int32) + _ROUND) & _MASK_HI
                cb_p[p * STAGE_ROWS + r, pl.ds(j * L, L)] = hi_b | lo_b
            return c2

        lax.fori_loop(0, STAGE_ROWS, pack_row, 0)
        return carry

    lax.fori_loop(0, CB_SIZE // STAGE_ROWS, stage_piece, 0)
    pltpu.sync_copy(idx_hbm.at[pl.ds(base, rows_per_w)], idx_v)

    def one_row(r, words_b, pooled_b, sem, first):
        row = base + r
        @pl.when(jnp.logical_not(first))
        def _():
            pltpu.make_async_copy(words_b, words_hbm.at[row], sem).wait()
            pltpu.make_async_copy(pooled_b, pooled_hbm.at[row], sem).wait()

        ivs = [idx_v[r, pl.ds(k * L, L)] for k in range(T_PAD // L)]
        ids = [ivs[t // L][t % L] for t in range(T)]

        @plsc.parallel_loop(0, NPC)
        def pc_body(pc):
            slp = pl.ds(pc * L, L)
            sums = []
            for w in range(NUM_WORDS):
                g0 = cb_p[ids[2 * w], slp]
                g1 = cb_p[ids[2 * w + 1], slp]
                a = plsc.bitcast(g0, jnp.bfloat16)
                b = plsc.bitcast(g1, jnp.bfloat16)
                sums.append(a + b)
            acc_lo = [None] * 4
            acc_hi = [None] * 4
            for w in range(NUM_WORDS):
                sb = plsc.bitcast(sums[w], jnp.int32)
                lo = plsc.bitcast(lax.shift_left(sb, 16), jnp.float32)
                hi = plsc.bitcast(sb & _MASK_HI, jnp.float32)
                words_b[w, pl.ds(pc * 2 * L, L)] = lo
                words_b[w, pl.ds(pc * 2 * L + L, L)] = hi
                k = w % 4
                acc_lo[k] = lo if acc_lo[k] is None else acc_lo[k] + lo
                acc_hi[k] = hi if acc_hi[k] is None else acc_hi[k] + hi
            pl_lo = ((acc_lo[0] + acc_lo[1]) + (acc_lo[2] + acc_lo[3]))
            pl_hi = ((acc_hi[0] + acc_hi[1]) + (acc_hi[2] + acc_hi[3]))
            pooled_b[pl.ds(pc * 2 * L, L)] = pl_lo * (1.0 / NUM_WORDS)
            pooled_b[pl.ds(pc * 2 * L + L, L)] = pl_hi * (1.0 / NUM_WORDS)

        pltpu.async_copy(words_b, words_hbm.at[row], sem)
        pltpu.async_copy(pooled_b, pooled_hbm.at[row], sem)

    def pair_body(i, carry):
        one_row(2 * i, words_v0, pooled_v0, sem0, i == 0)
        one_row(2 * i + 1, words_v1, pooled_v1, sem1, i == 0)
        return carry

    lax.fori_loop(0, rows_per_w // 2, pair_body, 0)
    last = base + rows_per_w - 2
    pltpu.make_async_copy(words_v0, words_hbm.at[last], sem0).wait()
    pltpu.make_async_copy(pooled_v0, pooled_hbm.at[last], sem0).wait()
    pltpu.make_async_copy(words_v1, words_hbm.at[last + 1], sem1).wait()
    pltpu.make_async_copy(pooled_v1, pooled_hbm.at[last + 1], sem1).wait()


def _sc_words(char_indices, char_codebook):
    nrows = char_indices.shape[0]
    mesh = plsc.VectorSubcoreMesh(core_axis_name="c", subcore_axis_name="s")
    f = pl.kernel(
        functools.partial(_sc_words_body, nrows),
        out_type=[
            jax.ShapeDtypeStruct((nrows, NUM_WORDS, CB_DIM), jnp.float32),
            jax.ShapeDtypeStruct((nrows, CB_DIM), jnp.float32),
        ],
        mesh=mesh,
        scratch_types=[
            pltpu.VMEM((CB_SIZE, CB_DIM // 2), jnp.int32),
            pltpu.VMEM((STAGE_ROWS, CB_DIM), jnp.float32),
            pltpu.VMEM((nrows // 32, T_PAD), jnp.int32),
            pltpu.VMEM((NUM_WORDS, CB_DIM), jnp.float32),
            pltpu.VMEM((NUM_WORDS, CB_DIM), jnp.float32),
            pltpu.VMEM((CB_DIM,), jnp.float32),
            pltpu.VMEM((CB_DIM,), jnp.float32),
            pltpu.SemaphoreType.DMA,
            pltpu.SemaphoreType.DMA,
        ],
        compiler_params=pltpu.CompilerParams(needs_layout_passes=False),
    )
    return f(char_indices, char_codebook)


def _tc_loss_body(pooled_ref, text_ref, w1_ref, b1_ref, w2_ref, b2_ref,
                  out_ref):
    pooled = pooled_ref[...]
    h = jnp.dot(pooled, w1_ref[...], preferred_element_type=jnp.float32)
    h = jnp.maximum(h + b1_ref[...][None, :], 0.0)
    hbar = jnp.mean(h, axis=0, keepdims=True)
    proj = jnp.dot(hbar, w2_ref[...], preferred_element_type=jnp.float32)
    proj = proj + b2_ref[...][None, :]
    tbar = jnp.mean(text_ref[...], axis=0, keepdims=True)
    d = proj - tbar
    out_ref[...] = jnp.reshape(jnp.mean(d * d), (1, 1))


def _tc_loss(pooled, text_embeddings, W1, b1, W2, b2):
    return pl.pallas_call(
        _tc_loss_body,
        out_shape=jax.ShapeDtypeStruct((1, 1), jnp.float32),
    )(pooled, text_embeddings, W1, b1, W2, b2)


def kernel(char_indices, text_embeddings, char_codebook, W1, b1, W2, b2):
    idx = jnp.pad(char_indices.astype(jnp.int32), ((0, 0), (0, T_PAD - T)))
    words, pooled = _sc_words(idx, char_codebook)
    loss = _tc_loss(pooled, text_embeddings, W1, b1, W2, b2)
    return words, loss[0, 0]
